# baseline (speedup 1.0000x reference)
import jax
import jax.numpy as jnp
from jax.experimental import pallas as pl

_TEMP = 50.0
_BC = 128
_LOG2E = 1.4426950408889634


def _fm_kernel(x_ref, y_ref, b_ref, choice_ref, fx_ref):
    y = y_ref[...]
    k = y.shape[0]
    c = jnp.sum(y * y, axis=1) + b_ref[0, :]
    cb = ((_TEMP * _LOG2E) * c)[None, :]
    yext = jnp.concatenate(
        [y, c[:, None], jnp.ones((k, 1), jnp.float32)], axis=1)

    def chunk(x):
        xy = jax.lax.dot_general(x, y, (((1,), (1,)), ((), ())),
                                 preferred_element_type=jnp.float32)
        u = (2.0 * _TEMP * _LOG2E) * xy - cb
        mu = jnp.max(u, axis=1, keepdims=True)
        e = jnp.exp2(u - mu)
        return jax.lax.dot_general(e, yext, (((1,), (0,)), ((), ())),
                                   preferred_element_type=jnp.float32)

    x = x_ref[...]
    n = x.shape[0]
    g = jnp.concatenate(
        [chunk(x[i * _BC:(i + 1) * _BC]) for i in range(n // _BC)],
        axis=0)
    gy = g[:, :-2]
    gc = g[:, -2]
    se = g[:, -1]
    x2 = jnp.sum(x * x, axis=1)
    choice_ref[...] = gy / se[:, None]
    fx_ref[...] = (2.0 * jnp.sum(gy * x, axis=1) - gc) / se - x2


def kernel(X, Y, intercept):
    N, D = X.shape
    K = Y.shape[1]
    choice, fx = pl.pallas_call(
        _fm_kernel,
        out_shape=[
            jax.ShapeDtypeStruct((N, D), jnp.float32),
            jax.ShapeDtypeStruct((N,), jnp.float32),
        ],
    )(X, Y[0], intercept)
    return choice, fx


# speedup vs baseline: 1.0611x; 1.0611x over previous
import jax
import jax.numpy as jnp
from jax.experimental import pallas as pl
from jax.experimental.pallas import tpu as pltpu

_TEMP = 50.0
_BN = 2048
_LOG2E = 1.4426950408889634


def _fm_kernel(x_ref, y_ref, b_ref, choice_ref, fx_ref, cb_ref, yext_ref):
    @pl.when(pl.program_id(0) == 0)
    def _init():
        y0 = y_ref[...]
        k = y0.shape[0]
        c = jnp.sum(y0 * y0, axis=1) + b_ref[0, :]
        cb_ref[...] = ((_TEMP * _LOG2E) * c)[None, :]
        yext_ref[...] = jnp.concatenate(
            [y0, c[:, None], jnp.ones((k, 1), jnp.float32)], axis=1)

    cb = cb_ref[...]
    y = y_ref[...]
    yext = yext_ref[...]

    def chunk(x):
        xy = jax.lax.dot_general(x, y, (((1,), (1,)), ((), ())),
                                 preferred_element_type=jnp.float32)
        u = (2.0 * _TEMP * _LOG2E) * xy - cb
        mu = jnp.max(u, axis=1, keepdims=True)
        e = jnp.exp2(u - mu)
        return jax.lax.dot_general(e, yext, (((1,), (0,)), ((), ())),
                                   preferred_element_type=jnp.float32)

    x = x_ref[...]
    h = x.shape[0] // 16
    g = jnp.concatenate([chunk(x[i * h:(i + 1) * h]) for i in range(16)],
                        axis=0)
    gy = g[:, :-2]
    gc = g[:, -2]
    se = g[:, -1]
    x2 = jnp.sum(x * x, axis=1)
    choice_ref[...] = gy / se[:, None]
    fx_ref[...] = (2.0 * jnp.sum(gy * x, axis=1) - gc) / se - x2


def kernel(X, Y, intercept):
    N, D = X.shape
    K = Y.shape[1]
    choice, fx = pl.pallas_call(
        _fm_kernel,
        grid=(N // _BN,),
        in_specs=[
            pl.BlockSpec((_BN, D), lambda i: (i, 0)),
            pl.BlockSpec((K, D), lambda i: (0, 0)),
            pl.BlockSpec((1, K), lambda i: (0, 0)),
        ],
        out_specs=[
            pl.BlockSpec((_BN, D), lambda i: (i, 0)),
            pl.BlockSpec((_BN,), lambda i: (i,)),
        ],
        out_shape=[
            jax.ShapeDtypeStruct((N, D), jnp.float32),
            jax.ShapeDtypeStruct((N,), jnp.float32),
        ],
        scratch_shapes=[
            pltpu.VMEM((1, K), jnp.float32),
            pltpu.VMEM((K, D + 2), jnp.float32),
        ],
    )(X, Y[0], intercept)
    return choice, fx


# speedup vs baseline: 1.0701x; 1.0085x over previous
import jax
import jax.numpy as jnp
from jax.experimental import pallas as pl
from jax.experimental.pallas import tpu as pltpu

_TEMP = 50.0
_BN = 2048
_LOG2E = 1.4426950408889634


def _fm_kernel(x_ref, y_ref, b_ref, choice_ref, fx_ref, cb_ref, yext_ref):
    @pl.when(pl.program_id(0) == 0)
    def _init():
        y0 = y_ref[...]
        k = y0.shape[0]
        c = jnp.sum(y0 * y0, axis=1) + b_ref[0, :]
        cb_ref[...] = ((_TEMP * _LOG2E) * c)[None, :]
        yext_ref[...] = jnp.concatenate(
            [y0, c[:, None], jnp.ones((k, 1), jnp.float32)], axis=1)

    cb = cb_ref[...]
    y = y_ref[...]
    yext = yext_ref[...]

    x = x_ref[...]
    h = x.shape[0] // 16

    def chunk(i):
        xc = x[i * h:(i + 1) * h]
        xy = jax.lax.dot_general(xc, y, (((1,), (1,)), ((), ())),
                                 preferred_element_type=jnp.float32)
        u = (2.0 * _TEMP * _LOG2E) * xy - cb
        mu = jnp.max(u, axis=1, keepdims=True)
        e = jnp.exp2(u - mu)
        g = jax.lax.dot_general(e, yext, (((1,), (0,)), ((), ())),
                                preferred_element_type=jnp.float32)
        gy = g[:, :-2]
        gc = g[:, -2]
        se = g[:, -1]
        x2 = jnp.sum(xc * xc, axis=1)
        choice_ref[pl.ds(i * h, h), :] = gy / se[:, None]
        fx_ref[pl.ds(i * h, h)] = (2.0 * jnp.sum(gy * xc, axis=1) - gc) / se - x2

    for i in range(16):
        chunk(i)


def kernel(X, Y, intercept):
    N, D = X.shape
    K = Y.shape[1]
    choice, fx = pl.pallas_call(
        _fm_kernel,
        grid=(N // _BN,),
        in_specs=[
            pl.BlockSpec((_BN, D), lambda i: (i, 0)),
            pl.BlockSpec((K, D), lambda i: (0, 0)),
            pl.BlockSpec((1, K), lambda i: (0, 0)),
        ],
        out_specs=[
            pl.BlockSpec((_BN, D), lambda i: (i, 0)),
            pl.BlockSpec((_BN,), lambda i: (i,)),
        ],
        out_shape=[
            jax.ShapeDtypeStruct((N, D), jnp.float32),
            jax.ShapeDtypeStruct((N,), jnp.float32),
        ],
        scratch_shapes=[
            pltpu.VMEM((1, K), jnp.float32),
            pltpu.VMEM((K, D + 2), jnp.float32),
        ],
    )(X, Y[0], intercept)
    return choice, fx
